# Initial kernel scaffold; baseline (speedup 1.0000x reference)
import jax
import jax.numpy as jnp
from jax.experimental import pallas as pl


def kernel(x, tables, W, b):
    raise NotImplementedError("write your pallas kernel here")


# speedup vs baseline: 8.0377x; 8.0377x over previous
import functools

import jax
import jax.numpy as jnp
from jax import lax
from jax.experimental import pallas as pl
from jax.experimental.pallas import tpu as pltpu
from jax.experimental.pallas import tpu_sc as plsc

N_FIELDS = 26
VOCAB = 100000
EMB_DIM = 32
BATCH = 16384
OUT_FEATURES = 128
IN_FEAT = N_FIELDS * EMB_DIM

_NW = 32
_ROWS = BATCH * N_FIELDS
_RPW = _ROWS // _NW
_G = 128
_NG = _RPW // _G
_NPAIR = _NG // 2


def _gather_body(tab_hbm, idx_hbm, out_hbm, idx_v, buf0, buf1, sem0, sem1):
    nc = lax.axis_size("c")
    wid = lax.axis_index("s") * nc + lax.axis_index("c")
    pltpu.sync_copy(idx_hbm.at[wid], idx_v)
    base = wid * _RPW

    pltpu.async_copy(tab_hbm.at[idx_v.at[0]], buf0, sem0)

    def body(i, carry):
        a = 2 * i
        pltpu.async_copy(tab_hbm.at[idx_v.at[a + 1]], buf1, sem1)
        pltpu.make_async_copy(tab_hbm.at[idx_v.at[a]], buf0, sem0).wait()
        pltpu.sync_copy(buf0, out_hbm.at[pl.ds(base + a * _G, _G)])

        @pl.when(i < _NPAIR - 1)
        def _():
            pltpu.async_copy(tab_hbm.at[idx_v.at[a + 2]], buf0, sem0)

        pltpu.make_async_copy(tab_hbm.at[idx_v.at[a + 1]], buf1, sem1).wait()
        pltpu.sync_copy(buf1, out_hbm.at[pl.ds(base + (a + 1) * _G, _G)])
        return carry

    lax.fori_loop(0, _NPAIR, body, 0)


@functools.partial(
    pl.kernel,
    out_type=jax.ShapeDtypeStruct((_ROWS, EMB_DIM), jnp.float32),
    mesh=plsc.VectorSubcoreMesh(core_axis_name="c", subcore_axis_name="s"),
    scratch_types=[
        pltpu.VMEM((_NG, _G), jnp.int32),
        pltpu.VMEM((_G, EMB_DIM), jnp.float32),
        pltpu.VMEM((_G, EMB_DIM), jnp.float32),
        pltpu.SemaphoreType.DMA,
        pltpu.SemaphoreType.DMA,
    ],
    compiler_params=pltpu.CompilerParams(use_tc_tiling_on_sc=False),
)
def _sc_gather(tab_hbm, idx_hbm, out_hbm, idx_v, buf0, buf1, sem0, sem1):
    _gather_body(tab_hbm, idx_hbm, out_hbm, idx_v, buf0, buf1, sem0, sem1)


def _mm_body(e_ref, w_ref, b_ref, o_ref):
    o_ref[...] = (
        jnp.dot(e_ref[...], w_ref[...], preferred_element_type=jnp.float32)
        + b_ref[...]
    )


_BM = 2048


def _tc_matmul(emb, W, b):
    return pl.pallas_call(
        _mm_body,
        grid=(BATCH // _BM,),
        in_specs=[
            pl.BlockSpec((_BM, IN_FEAT), lambda i: (i, 0)),
            pl.BlockSpec((IN_FEAT, OUT_FEATURES), lambda i: (0, 0)),
            pl.BlockSpec((1, OUT_FEATURES), lambda i: (0, 0)),
        ],
        out_specs=pl.BlockSpec((_BM, OUT_FEATURES), lambda i: (i, 0)),
        out_shape=jax.ShapeDtypeStruct((BATCH, OUT_FEATURES), jnp.float32),
    )(emb, W, b.reshape(1, OUT_FEATURES))


def kernel(x, tables, W, b):
    tab_flat = tables.reshape(N_FIELDS * VOCAB, EMB_DIM)
    idx = x.astype(jnp.int32) + (
        jnp.arange(N_FIELDS, dtype=jnp.int32) * VOCAB
    )
    idx = idx.reshape(_NW, _NG, _G)
    emb = _sc_gather(tab_flat, idx)
    emb = emb.reshape(BATCH, IN_FEAT)
    return _tc_matmul(emb, W, b)


# speedup vs baseline: 11.6233x; 1.4461x over previous
import functools

import jax
import jax.numpy as jnp
from jax import lax
from jax.experimental import pallas as pl
from jax.experimental.pallas import tpu as pltpu
from jax.experimental.pallas import tpu_sc as plsc

N_FIELDS = 26
VOCAB = 100000
EMB_DIM = 32
BATCH = 16384
OUT_FEATURES = 128
IN_FEAT = N_FIELDS * EMB_DIM

_NW = 32
_ROWS = BATCH * N_FIELDS
_RPW = _ROWS // _NW
_G = 128
_NG = _RPW // _G
_NPAIR = _NG // 2


def _gather_body(tab_hbm, idx_hbm, out_hbm, idx_v, buf0, buf1, sem0, sem1):
    nc = lax.axis_size("c")
    wid = lax.axis_index("s") * nc + lax.axis_index("c")
    pltpu.sync_copy(idx_hbm.at[wid], idx_v)
    base = wid * _RPW

    pltpu.async_copy(tab_hbm.at[idx_v.at[0]], buf0, sem0)

    def body(i, carry):
        a = 2 * i
        pltpu.async_copy(tab_hbm.at[idx_v.at[a + 1]], buf1, sem1)
        pltpu.make_async_copy(tab_hbm.at[idx_v.at[a]], buf0, sem0).wait()
        pltpu.sync_copy(buf0, out_hbm.at[pl.ds(base + a * _G, _G)])

        @pl.when(i < _NPAIR - 1)
        def _():
            pltpu.async_copy(tab_hbm.at[idx_v.at[a + 2]], buf0, sem0)

        pltpu.make_async_copy(tab_hbm.at[idx_v.at[a + 1]], buf1, sem1).wait()
        pltpu.sync_copy(buf1, out_hbm.at[pl.ds(base + (a + 1) * _G, _G)])
        return carry

    lax.fori_loop(0, _NPAIR, body, 0)


@functools.partial(
    pl.kernel,
    out_type=jax.ShapeDtypeStruct((_ROWS, EMB_DIM), jnp.float32),
    mesh=plsc.VectorSubcoreMesh(core_axis_name="c", subcore_axis_name="s"),
    scratch_types=[
        pltpu.VMEM((_NG, _G), jnp.int32),
        pltpu.VMEM((_G, EMB_DIM), jnp.float32),
        pltpu.VMEM((_G, EMB_DIM), jnp.float32),
        pltpu.SemaphoreType.DMA,
        pltpu.SemaphoreType.DMA,
    ],
    compiler_params=pltpu.CompilerParams(use_tc_tiling_on_sc=False),
)
def _sc_gather(tab_hbm, idx_hbm, out_hbm, idx_v, buf0, buf1, sem0, sem1):
    _gather_body(tab_hbm, idx_hbm, out_hbm, idx_v, buf0, buf1, sem0, sem1)


_VQ = VOCAB // 4


_TR_CHUNK = 3125


def _tr_body(in_ref, out_ref):
    for k in range(_VQ // _TR_CHUNK):
        for a in range(4):
            lo = _VQ * a + _TR_CHUNK * k
            out_ref[_TR_CHUNK * k:_TR_CHUNK * (k + 1),
                    32 * a:32 * (a + 1)] = in_ref[0, :, lo:lo + _TR_CHUNK].T


def _tc_transpose(tabT):
    rows_per_field = _VQ
    return pl.pallas_call(
        _tr_body,
        grid=(N_FIELDS,),
        in_specs=[pl.BlockSpec((1, EMB_DIM, VOCAB), lambda f: (f, 0, 0))],
        out_specs=pl.BlockSpec((rows_per_field, 128), lambda f: (f, 0)),
        out_shape=jax.ShapeDtypeStruct((N_FIELDS * rows_per_field, 128),
                                       jnp.float32),
    )(tabT)


def _mm_body(e_ref, w_ref, b_ref, o_ref):
    o_ref[...] = (
        jnp.dot(e_ref[...], w_ref[...], preferred_element_type=jnp.float32)
        + b_ref[...]
    )


_BM = 2048


def _tc_matmul(emb, W, b):
    return pl.pallas_call(
        _mm_body,
        grid=(BATCH // _BM,),
        in_specs=[
            pl.BlockSpec((_BM, IN_FEAT), lambda i: (i, 0)),
            pl.BlockSpec((IN_FEAT, OUT_FEATURES), lambda i: (0, 0)),
            pl.BlockSpec((1, OUT_FEATURES), lambda i: (0, 0)),
        ],
        out_specs=pl.BlockSpec((_BM, OUT_FEATURES), lambda i: (i, 0)),
        out_shape=jax.ShapeDtypeStruct((BATCH, OUT_FEATURES), jnp.float32),
    )(emb, W, b.reshape(1, OUT_FEATURES))


def kernel(x, tables, W, b):
    tabT = jnp.transpose(tables, (0, 2, 1))
    tab_flat = _tc_transpose(tabT).reshape(N_FIELDS * VOCAB, EMB_DIM)
    xi = x.astype(jnp.int32)
    idx = (
        jnp.arange(N_FIELDS, dtype=jnp.int32) * VOCAB
        + (xi % _VQ) * 4
        + xi // _VQ
    )
    idx = idx.reshape(_NW, _NG, _G)
    emb = _sc_gather(tab_flat, idx)
    emb = emb.reshape(BATCH, IN_FEAT)
    return _tc_matmul(emb, W, b)


# speedup vs baseline: 14.0842x; 1.2117x over previous
import functools

import jax
import jax.numpy as jnp
from jax import lax
from jax.experimental import pallas as pl
from jax.experimental.pallas import tpu as pltpu
from jax.experimental.pallas import tpu_sc as plsc

N_FIELDS = 26
VOCAB = 100000
EMB_DIM = 32
BATCH = 16384
OUT_FEATURES = 128
IN_FEAT = N_FIELDS * EMB_DIM

_NW = 32
_ROWS = BATCH * N_FIELDS
_RPW = _ROWS // _NW
_G = 128
_NG = _RPW // _G
_NPAIR = _NG // 2


def _gather_body(tab_hbm, idx_hbm, out_hbm, idx_v, buf0, buf1, sem0, sem1):
    nc = lax.axis_size("c")
    wid = lax.axis_index("s") * nc + lax.axis_index("c")
    pltpu.sync_copy(idx_hbm.at[wid], idx_v)
    base = wid * _RPW

    pltpu.async_copy(tab_hbm.at[idx_v.at[0]], buf0, sem0)

    def body(i, carry):
        a = 2 * i
        pltpu.async_copy(tab_hbm.at[idx_v.at[a + 1]], buf1, sem1)
        pltpu.make_async_copy(tab_hbm.at[idx_v.at[a]], buf0, sem0).wait()
        pltpu.sync_copy(buf0, out_hbm.at[pl.ds(base + a * _G, _G)])

        @pl.when(i < _NPAIR - 1)
        def _():
            pltpu.async_copy(tab_hbm.at[idx_v.at[a + 2]], buf0, sem0)

        pltpu.make_async_copy(tab_hbm.at[idx_v.at[a + 1]], buf1, sem1).wait()
        pltpu.sync_copy(buf1, out_hbm.at[pl.ds(base + (a + 1) * _G, _G)])
        return carry

    lax.fori_loop(0, _NPAIR, body, 0)


@functools.partial(
    pl.kernel,
    out_type=jax.ShapeDtypeStruct((_ROWS, EMB_DIM), jnp.float32),
    mesh=plsc.VectorSubcoreMesh(core_axis_name="c", subcore_axis_name="s"),
    scratch_types=[
        pltpu.VMEM((_NG, _G), jnp.int32),
        pltpu.VMEM((_G, EMB_DIM), jnp.float32),
        pltpu.VMEM((_G, EMB_DIM), jnp.float32),
        pltpu.SemaphoreType.DMA,
        pltpu.SemaphoreType.DMA,
    ],
    compiler_params=pltpu.CompilerParams(use_tc_tiling_on_sc=False),
)
def _sc_gather(tab_hbm, idx_hbm, out_hbm, idx_v, buf0, buf1, sem0, sem1):
    _gather_body(tab_hbm, idx_hbm, out_hbm, idx_v, buf0, buf1, sem0, sem1)


_VQ = VOCAB // 4


_NT = VOCAB // 512


def _tr_body(in_ref, out_ref):
    ident = jnp.eye(128, dtype=jnp.float32)
    dn = (((0,), (0,)), ((), ()))

    def body(i, carry):
        for u in range(2):
            t = 2 * i + u
            base = 512 * t
            xs = jnp.concatenate(
                [in_ref[0, :, pl.ds(base + 128 * a, 128)] for a in range(4)],
                axis=0,
            )
            out_ref[pl.ds(128 * t, 128), :] = lax.dot_general(
                xs, ident, dn, preferred_element_type=jnp.float32
            )
        return carry

    lax.fori_loop(0, _NT // 2, body, 0)
    for t in range(2 * (_NT // 2), _NT):
        base = 512 * t
        xs = jnp.concatenate(
            [in_ref[0, :, base + 128 * a:base + 128 * (a + 1)]
             for a in range(4)],
            axis=0,
        )
        out_ref[128 * t:128 * (t + 1), :] = lax.dot_general(
            xs, ident, dn, preferred_element_type=jnp.float32
        )
    tb = 512 * _NT
    xt = jnp.concatenate(
        [in_ref[0, :, tb + 40 * a:tb + 40 * (a + 1)] for a in range(4)],
        axis=0,
    )
    out_ref[128 * _NT:_VQ, :] = lax.dot_general(
        xt, ident, dn, preferred_element_type=jnp.float32
    )


def _tc_transpose(tabT):
    rows_per_field = _VQ
    return pl.pallas_call(
        _tr_body,
        grid=(N_FIELDS,),
        in_specs=[pl.BlockSpec((1, EMB_DIM, VOCAB), lambda f: (f, 0, 0))],
        out_specs=pl.BlockSpec((rows_per_field, 128), lambda f: (f, 0)),
        out_shape=jax.ShapeDtypeStruct((N_FIELDS * rows_per_field, 128),
                                       jnp.float32),
    )(tabT)


def _mm_body(e_ref, w_ref, b_ref, o_ref):
    o_ref[...] = (
        jnp.dot(e_ref[...], w_ref[...], preferred_element_type=jnp.float32)
        + b_ref[...]
    )


_BM = 2048


def _tc_matmul(emb, W, b):
    return pl.pallas_call(
        _mm_body,
        grid=(BATCH // _BM,),
        in_specs=[
            pl.BlockSpec((_BM, IN_FEAT), lambda i: (i, 0)),
            pl.BlockSpec((IN_FEAT, OUT_FEATURES), lambda i: (0, 0)),
            pl.BlockSpec((1, OUT_FEATURES), lambda i: (0, 0)),
        ],
        out_specs=pl.BlockSpec((_BM, OUT_FEATURES), lambda i: (i, 0)),
        out_shape=jax.ShapeDtypeStruct((BATCH, OUT_FEATURES), jnp.float32),
    )(emb, W, b.reshape(1, OUT_FEATURES))


def kernel(x, tables, W, b):
    tabT = jnp.transpose(tables, (0, 2, 1))
    tab_flat = _tc_transpose(tabT).reshape(N_FIELDS * VOCAB, EMB_DIM)
    xi = x.astype(jnp.int32)
    vt = xi - 512 * _NT
    r_main = 128 * (xi // 512) + xi % 128
    a_main = (xi // 128) % 4
    r_tail = 128 * _NT + vt % 40
    a_tail = vt // 40
    tail = xi >= 512 * _NT
    r = jnp.where(tail, r_tail, r_main)
    a = jnp.where(tail, a_tail, a_main)
    idx = jnp.arange(N_FIELDS, dtype=jnp.int32) * VOCAB + r * 4 + a
    idx = idx.reshape(_NW, _NG, _G)
    emb = _sc_gather(tab_flat, idx)
    emb = emb.reshape(BATCH, IN_FEAT)
    return _tc_matmul(emb, W, b)


# speedup vs baseline: 25.9614x; 1.8433x over previous
import functools

import jax
import jax.numpy as jnp
from jax import lax
from jax.experimental import pallas as pl
from jax.experimental.pallas import tpu as pltpu
from jax.experimental.pallas import tpu_sc as plsc

N_FIELDS = 26
VOCAB = 100000
EMB_DIM = 32
BATCH = 16384
OUT_FEATURES = 128
IN_FEAT = N_FIELDS * EMB_DIM

_NW = 32
_ROWS = BATCH * N_FIELDS
_RPW = _ROWS // _NW
_G = 128
_NG = _RPW // _G
_NPAIR = _NG // 2


def _gather_body(tab_hbm, idx_hbm, out_hbm, idx_v, buf0, buf1, sem0, sem1):
    nc = lax.axis_size("c")
    wid = lax.axis_index("s") * nc + lax.axis_index("c")
    pltpu.sync_copy(idx_hbm.at[wid], idx_v)
    base = wid * _RPW

    pltpu.async_copy(tab_hbm.at[idx_v.at[0]], buf0, sem0)

    def body(i, carry):
        a = 2 * i
        pltpu.async_copy(tab_hbm.at[idx_v.at[a + 1]], buf1, sem1)
        pltpu.make_async_copy(tab_hbm.at[idx_v.at[a]], buf0, sem0).wait()
        pltpu.sync_copy(buf0, out_hbm.at[pl.ds(base + a * _G, _G)])

        @pl.when(i < _NPAIR - 1)
        def _():
            pltpu.async_copy(tab_hbm.at[idx_v.at[a + 2]], buf0, sem0)

        pltpu.make_async_copy(tab_hbm.at[idx_v.at[a + 1]], buf1, sem1).wait()
        pltpu.sync_copy(buf1, out_hbm.at[pl.ds(base + (a + 1) * _G, _G)])
        return carry

    lax.fori_loop(0, _NPAIR, body, 0)


@functools.partial(
    pl.kernel,
    out_type=jax.ShapeDtypeStruct((_ROWS, EMB_DIM), jnp.float32),
    mesh=plsc.VectorSubcoreMesh(core_axis_name="c", subcore_axis_name="s"),
    scratch_types=[
        pltpu.VMEM((_NG, _G), jnp.int32),
        pltpu.VMEM((_G, EMB_DIM), jnp.float32),
        pltpu.VMEM((_G, EMB_DIM), jnp.float32),
        pltpu.SemaphoreType.DMA,
        pltpu.SemaphoreType.DMA,
    ],
    compiler_params=pltpu.CompilerParams(use_tc_tiling_on_sc=False),
)
def _sc_gather(tab_hbm, idx_hbm, out_hbm, idx_v, buf0, buf1, sem0, sem1):
    _gather_body(tab_hbm, idx_hbm, out_hbm, idx_v, buf0, buf1, sem0, sem1)


_VQ = VOCAB // 4


_NT = VOCAB // 512


def _tr_body(in_ref, out_ref):
    ident = jnp.eye(128, dtype=jnp.float32)
    dn = (((0,), (0,)), ((), ()))

    def body(i, carry):
        for u in range(8):
            t = 8 * i + u
            base = 512 * t
            xs = jnp.concatenate(
                [in_ref[0, :, pl.ds(base + 128 * a, 128)] for a in range(4)],
                axis=0,
            )
            out_ref[pl.ds(128 * t, 128), :] = lax.dot_general(
                xs, ident, dn, preferred_element_type=jnp.float32
            )
        return carry

    lax.fori_loop(0, _NT // 8, body, 0)
    for t in range(8 * (_NT // 8), _NT):
        base = 512 * t
        xs = jnp.concatenate(
            [in_ref[0, :, base + 128 * a:base + 128 * (a + 1)]
             for a in range(4)],
            axis=0,
        )
        out_ref[128 * t:128 * (t + 1), :] = lax.dot_general(
            xs, ident, dn, preferred_element_type=jnp.float32
        )
    tb = 512 * _NT
    xt = jnp.concatenate(
        [in_ref[0, :, tb + 40 * a:tb + 40 * (a + 1)] for a in range(4)],
        axis=0,
    )
    out_ref[128 * _NT:_VQ, :] = lax.dot_general(
        xt, ident, dn, preferred_element_type=jnp.float32
    )


def _tc_transpose(tabT):
    rows_per_field = _VQ
    return pl.pallas_call(
        _tr_body,
        grid=(N_FIELDS,),
        in_specs=[pl.BlockSpec((1, EMB_DIM, VOCAB), lambda f: (f, 0, 0))],
        out_specs=pl.BlockSpec((rows_per_field, 128), lambda f: (f, 0)),
        out_shape=jax.ShapeDtypeStruct((N_FIELDS * rows_per_field, 128),
                                       jnp.float32),
    )(tabT)


def _mm_body(e_ref, w_ref, b_ref, o_ref):
    o_ref[...] = (
        jnp.dot(e_ref[...], w_ref[...], preferred_element_type=jnp.float32)
        + b_ref[...]
    )


_BM = 2048


def _tc_matmul(emb, W, b):
    return pl.pallas_call(
        _mm_body,
        grid=(BATCH // _BM,),
        in_specs=[
            pl.BlockSpec((_BM, IN_FEAT), lambda i: (i, 0)),
            pl.BlockSpec((IN_FEAT, OUT_FEATURES), lambda i: (0, 0)),
            pl.BlockSpec((1, OUT_FEATURES), lambda i: (0, 0)),
        ],
        out_specs=pl.BlockSpec((_BM, OUT_FEATURES), lambda i: (i, 0)),
        out_shape=jax.ShapeDtypeStruct((BATCH, OUT_FEATURES), jnp.float32),
    )(emb, W, b.reshape(1, OUT_FEATURES))


def kernel(x, tables, W, b):
    tabT = jnp.transpose(tables, (0, 2, 1))
    tab_flat = _tc_transpose(tabT).reshape(N_FIELDS * VOCAB, EMB_DIM)
    xi = x.astype(jnp.int32)
    vt = xi - 512 * _NT
    r_main = 128 * (xi // 512) + xi % 128
    a_main = (xi // 128) % 4
    r_tail = 128 * _NT + vt % 40
    a_tail = vt // 40
    tail = xi >= 512 * _NT
    r = jnp.where(tail, r_tail, r_main)
    a = jnp.where(tail, a_tail, a_main)
    idx = jnp.arange(N_FIELDS, dtype=jnp.int32) * VOCAB + r * 4 + a
    idx = idx.reshape(_NW, _NG, _G)
    emb = _sc_gather(tab_flat, idx)
    emb = emb.reshape(BATCH, IN_FEAT)
    return _tc_matmul(emb, W, b)


# speedup vs baseline: 26.5976x; 1.0245x over previous
import functools

import jax
import jax.numpy as jnp
from jax import lax
from jax.experimental import pallas as pl
from jax.experimental.pallas import tpu as pltpu
from jax.experimental.pallas import tpu_sc as plsc

N_FIELDS = 26
VOCAB = 100000
EMB_DIM = 32
BATCH = 16384
OUT_FEATURES = 128
IN_FEAT = N_FIELDS * EMB_DIM

_NW = 32
_ROWS = BATCH * N_FIELDS
_RPW = _ROWS // _NW
_G = 128
_NG = _RPW // _G
_NPAIR = _NG // 2


def _gather_body(tab_hbm, idx_hbm, out_hbm, idx_v, buf0, buf1, sem0, sem1):
    nc = lax.axis_size("c")
    wid = lax.axis_index("s") * nc + lax.axis_index("c")
    pltpu.sync_copy(idx_hbm.at[wid], idx_v)
    base = wid * _RPW

    pltpu.async_copy(tab_hbm.at[idx_v.at[0]], buf0, sem0)

    def body(i, carry):
        a = 2 * i
        pltpu.async_copy(tab_hbm.at[idx_v.at[a + 1]], buf1, sem1)
        pltpu.make_async_copy(tab_hbm.at[idx_v.at[a]], buf0, sem0).wait()
        pltpu.sync_copy(buf0, out_hbm.at[pl.ds(base + a * _G, _G)])

        @pl.when(i < _NPAIR - 1)
        def _():
            pltpu.async_copy(tab_hbm.at[idx_v.at[a + 2]], buf0, sem0)

        pltpu.make_async_copy(tab_hbm.at[idx_v.at[a + 1]], buf1, sem1).wait()
        pltpu.sync_copy(buf1, out_hbm.at[pl.ds(base + (a + 1) * _G, _G)])
        return carry

    lax.fori_loop(0, _NPAIR, body, 0)


@functools.partial(
    pl.kernel,
    out_type=jax.ShapeDtypeStruct((_ROWS, EMB_DIM), jnp.float32),
    mesh=plsc.VectorSubcoreMesh(core_axis_name="c", subcore_axis_name="s"),
    scratch_types=[
        pltpu.VMEM((_NG, _G), jnp.int32),
        pltpu.VMEM((_G, EMB_DIM), jnp.float32),
        pltpu.VMEM((_G, EMB_DIM), jnp.float32),
        pltpu.SemaphoreType.DMA,
        pltpu.SemaphoreType.DMA,
    ],
    compiler_params=pltpu.CompilerParams(use_tc_tiling_on_sc=False),
)
def _sc_gather(tab_hbm, idx_hbm, out_hbm, idx_v, buf0, buf1, sem0, sem1):
    _gather_body(tab_hbm, idx_hbm, out_hbm, idx_v, buf0, buf1, sem0, sem1)


_VQ = VOCAB // 4


_NT = VOCAB // 512


def _tr_body(in_ref, out_ref):
    ident = jnp.eye(128, dtype=jnp.float32)
    dn = (((0,), (0,)), ((), ()))

    def body(i, carry):
        for u in range(14):
            t = 14 * i + u
            base = 512 * t
            xs = jnp.concatenate(
                [in_ref[0, :, pl.ds(base + 128 * a, 128)] for a in range(4)],
                axis=0,
            )
            out_ref[pl.ds(128 * t, 128), :] = lax.dot_general(
                xs, ident, dn, preferred_element_type=jnp.float32
            )
        return carry

    lax.fori_loop(0, _NT // 14, body, 0)
    for t in range(14 * (_NT // 14), _NT):
        base = 512 * t
        xs = jnp.concatenate(
            [in_ref[0, :, base + 128 * a:base + 128 * (a + 1)]
             for a in range(4)],
            axis=0,
        )
        out_ref[128 * t:128 * (t + 1), :] = lax.dot_general(
            xs, ident, dn, preferred_element_type=jnp.float32
        )
    tb = 512 * _NT
    xt = jnp.concatenate(
        [in_ref[0, :, tb + 40 * a:tb + 40 * (a + 1)] for a in range(4)],
        axis=0,
    )
    out_ref[128 * _NT:_VQ, :] = lax.dot_general(
        xt, ident, dn, preferred_element_type=jnp.float32
    )


def _tc_transpose(tabT):
    rows_per_field = _VQ
    return pl.pallas_call(
        _tr_body,
        grid=(N_FIELDS,),
        in_specs=[pl.BlockSpec((1, EMB_DIM, VOCAB), lambda f: (f, 0, 0))],
        out_specs=pl.BlockSpec((rows_per_field, 128), lambda f: (f, 0)),
        out_shape=jax.ShapeDtypeStruct((N_FIELDS * rows_per_field, 128),
                                       jnp.float32),
    )(tabT)


def _mm_body(e_ref, w_ref, b_ref, o_ref):
    o_ref[...] = (
        jnp.dot(e_ref[...], w_ref[...], preferred_element_type=jnp.float32)
        + b_ref[...]
    )


_BM = 2048


def _tc_matmul(emb, W, b):
    return pl.pallas_call(
        _mm_body,
        grid=(BATCH // _BM,),
        in_specs=[
            pl.BlockSpec((_BM, IN_FEAT), lambda i: (i, 0)),
            pl.BlockSpec((IN_FEAT, OUT_FEATURES), lambda i: (0, 0)),
            pl.BlockSpec((1, OUT_FEATURES), lambda i: (0, 0)),
        ],
        out_specs=pl.BlockSpec((_BM, OUT_FEATURES), lambda i: (i, 0)),
        out_shape=jax.ShapeDtypeStruct((BATCH, OUT_FEATURES), jnp.float32),
    )(emb, W, b.reshape(1, OUT_FEATURES))


def kernel(x, tables, W, b):
    tabT = jnp.transpose(tables, (0, 2, 1))
    tab_flat = _tc_transpose(tabT).reshape(N_FIELDS * VOCAB, EMB_DIM)
    xi = x.astype(jnp.int32)
    vt = xi - 512 * _NT
    r_main = 128 * (xi // 512) + xi % 128
    a_main = (xi // 128) % 4
    r_tail = 128 * _NT + vt % 40
    a_tail = vt // 40
    tail = xi >= 512 * _NT
    r = jnp.where(tail, r_tail, r_main)
    a = jnp.where(tail, a_tail, a_main)
    idx = jnp.arange(N_FIELDS, dtype=jnp.int32) * VOCAB + r * 4 + a
    idx = idx.reshape(_NW, _NG, _G)
    emb = _sc_gather(tab_flat, idx)
    emb = emb.reshape(BATCH, IN_FEAT)
    return _tc_matmul(emb, W, b)
